# baseline (speedup 1.0000x reference)
import functools

import jax
import jax.numpy as jnp
from jax import lax
from jax.experimental import pallas as pl
from jax.experimental.pallas import tpu as pltpu
from jax.experimental.pallas import tpu_sc as plsc

C = 64
NX = 432
NY = 496
B = 4
M = 48000
S = NX * NY
ROWS_BIG = 56
ROWS_SMALL = 48
WCELLS = ROWS_BIG * NY
FB = 376
MP = FB * 128
STR_ROWS = 4
STRIPE = STR_ROWS * NY
NSTR = ROWS_BIG // STR_ROWS
MV = M // 16

_BLK = 9600


def _prep_body(feats_ref, x_ref, y_ref, s_ref, featsT_ref, dst_ref, cnt_ref):
    f = feats_ref[...]
    featsT_ref[...] = f.T

    @pl.when(pl.program_id(0) == 0)
    def _():
        s = s_ref[...]
        dst = s * S + (NX - 1 - x_ref[...]) * NY + y_ref[...]
        dst_ref[...] = dst
        lane = lax.broadcasted_iota(jnp.int32, (1, 128), 1)
        cnt = jnp.zeros((1, 128), jnp.int32)
        for bb in range(B):
            nb = jnp.sum((s == bb).astype(jnp.int32))
            cnt = cnt + jnp.where(lane == bb, nb, 0)
        cnt_ref[...] = cnt


def _prep(feats, x2d, y2d, s2d):
    return pl.pallas_call(
        _prep_body,
        grid=(M // _BLK,),
        in_specs=[
            pl.BlockSpec((_BLK, C), lambda i: (i, 0)),
            pl.BlockSpec((M // 128, 128), lambda i: (0, 0)),
            pl.BlockSpec((M // 128, 128), lambda i: (0, 0)),
            pl.BlockSpec((M // 128, 128), lambda i: (0, 0)),
        ],
        out_specs=[
            pl.BlockSpec((C, _BLK), lambda i: (0, i)),
            pl.BlockSpec((M // 128, 128), lambda i: (0, 0)),
            pl.BlockSpec((1, 128), lambda i: (0, 0)),
        ],
        out_shape=[
            jax.ShapeDtypeStruct((C, M), jnp.float32),
            jax.ShapeDtypeStruct((M // 128, 128), jnp.int32),
            jax.ShapeDtypeStruct((1, 128), jnp.int32),
        ],
    )(feats, x2d, y2d, s2d)


@functools.partial(
    pl.kernel,
    mesh=plsc.VectorSubcoreMesh(core_axis_name="c", subcore_axis_name="s"),
    out_type=jax.ShapeDtypeStruct((B, C, NX, NY), jnp.float32),
    compiler_params=pltpu.CompilerParams(
        needs_layout_passes=False, use_tc_tiling_on_sc=True
    ),
    scratch_types=[
        pltpu.VMEM((WCELLS,), jnp.int32),
        pltpu.VMEM((2, STR_ROWS, NY), jnp.float32),
        pltpu.VMEM((1, 128), jnp.int32),
        pltpu.SemaphoreType.DMA,
        pltpu.SemaphoreType.DMA,
        pltpu.SemaphoreType.DMA,
    ],
)
def _sc_scatter(
    featsT_hbm, dst_hbm, cnt_hbm, out_hbm,
    win_v, out_v, cnt_v, sem_f, sem_o0, sem_o1,
):
    wid = lax.axis_index("s") * 2 + lax.axis_index("c")
    b = wid // 8
    j = wid % 8
    rs = j * ROWS_BIG - 8 * jnp.maximum(0, j - 6)
    nrows = ROWS_BIG - 8 * (j >= 6).astype(jnp.int32)
    base = (b * NX + rs) * NY
    ncell = nrows * NY
    nstr = nrows // STR_ROWS

    fill = jnp.full((16,), M, jnp.int32)

    def init_body(i, _):
        for k in range(8):
            win_v[pl.ds((i * 8 + k) * 16, 16)] = fill
        return 0

    lax.fori_loop(0, WCELLS // 128, init_body, 0)

    pltpu.sync_copy(cnt_hbm, cnt_v)
    cvec = cnt_v[0, pl.ds(0, 16)]
    iota = lax.iota(jnp.int32, 16)
    lo = jnp.sum(jnp.where(iota < b, cvec, 0))
    nb = jnp.sum(jnp.where(iota == b, cvec, 0))
    v8_lo = lo // 128
    v8_hi = (lo + nb + 127) // 128

    def phase1(dst_v):
        pltpu.sync_copy(dst_hbm, dst_v)

        def body(v8, _):
            for k in range(8):
                idx = dst_v[v8, pl.ds(k * 16, 16)]
                m = iota + v8 * 128 + k * 16
                local = idx - base
                mask = (local >= 0) & (local < ncell)
                safe = jnp.where(mask, local, 0)
                plsc.store_scatter(win_v, [safe], m, mask=mask)
            return 0

        lax.fori_loop(v8_lo, v8_hi, body, 0)

    pl.run_scoped(phase1, pltpu.VMEM((M // 128, 128), jnp.int32))

    def phase2(feat_v):
      pltpu.make_async_copy(
          featsT_hbm.at[pl.ds(0, FB)], feat_v.at[0], sem_f
      ).start()

      def chan(c, _):
          par = lax.rem(c, 2)
          pltpu.make_async_copy(
              featsT_hbm.at[pl.ds(c * FB, FB)], feat_v.at[par], sem_f
          ).wait()

          @pl.when(c < C - 1)
          def _():
              pltpu.make_async_copy(
                  featsT_hbm.at[pl.ds((c + 1) * FB, FB)], feat_v.at[1 - par], sem_f
              ).start()

          prow = jnp.zeros((16,), jnp.int32) + par

          for st in range(NSTR):
              p = st % 2
              sem_o = sem_o0 if p == 0 else sem_o1

              @pl.when(st < nstr)
              def _():
                  src = out_v.at[p]
                  if st >= 2:
                      pltpu.make_async_copy(
                          src,
                          out_hbm.at[b, c, pl.ds(rs + (st - 2) * STR_ROWS, STR_ROWS), :],
                          sem_o,
                      ).wait()
                  else:

                      @pl.when(c > 0)
                      def _():
                          pltpu.make_async_copy(
                              src, out_hbm.at[b, c, pl.ds(rs, STR_ROWS), :], sem_o
                          ).wait()

                  for r in range(STR_ROWS):
                      lrow = st * STR_ROWS + r

                      @plsc.parallel_loop(0, NY, step=16, unroll=8)
                      def _(i):
                          widx = win_v[pl.ds(lrow * NY + i, 16)]
                          vals = plsc.load_gather(
                              feat_v,
                              [prow, lax.shift_right_logical(widx, 7), widx & 127],
                          )
                          out_v[p, r, pl.ds(i, 16)] = vals

                  pltpu.make_async_copy(
                      src,
                      out_hbm.at[b, c, pl.ds(rs + st * STR_ROWS, STR_ROWS), :],
                      sem_o,
                  ).start()

          return 0

      lax.fori_loop(0, C, chan, 0)

      for p, sem_o in ((0, sem_o0), (1, sem_o1)):
          pltpu.make_async_copy(
              out_v.at[p], out_hbm.at[0, 0, pl.ds(p * STR_ROWS, STR_ROWS), :], sem_o
          ).wait()

    pl.run_scoped(phase2, pltpu.VMEM((2, FB, 128), jnp.float32))


def kernel(batch_pillar_features, batch_indices, sample_indices, batch_size):
    del batch_size
    featsT, dst2d, cnt = _prep(
        batch_pillar_features,
        batch_indices[:, 2].reshape(M // 128, 128),
        batch_indices[:, 1].reshape(M // 128, 128),
        sample_indices.reshape(M // 128, 128),
    )
    featsT_p = jnp.pad(
        featsT.reshape(C, M // 128, 128), ((0, 0), (0, 1), (0, 0))
    ).reshape(C * FB, 128)
    return _sc_scatter(featsT_p, dst2d, cnt)


# speedup vs baseline: 1.0111x; 1.0111x over previous
import functools

import jax
import jax.numpy as jnp
from jax import lax
from jax.experimental import pallas as pl
from jax.experimental.pallas import tpu as pltpu
from jax.experimental.pallas import tpu_sc as plsc

C = 64
NX = 432
NY = 496
B = 4
M = 48000
S = NX * NY
ROWS_BIG = 56
ROWS_SMALL = 48
WCELLS = ROWS_BIG * NY
FB = 376
MP = FB * 128
STR_ROWS = 4
STRIPE = STR_ROWS * NY
NSTR = ROWS_BIG // STR_ROWS
MV = M // 16

_BLK = 9600


def _prep_body(feats_ref, x_ref, y_ref, s_ref, featsT_ref, dst_ref, cnt_ref):
    f = feats_ref[...]
    featsT_ref[...] = f.T

    @pl.when(pl.program_id(0) == 0)
    def _():
        s = s_ref[...]
        dst = s * S + (NX - 1 - x_ref[...]) * NY + y_ref[...]
        dst_ref[...] = dst
        lane = lax.broadcasted_iota(jnp.int32, (1, 128), 1)
        cnt = jnp.zeros((1, 128), jnp.int32)
        for bb in range(B):
            nb = jnp.sum((s == bb).astype(jnp.int32))
            cnt = cnt + jnp.where(lane == bb, nb, 0)
        cnt_ref[...] = cnt


def _prep(feats, x2d, y2d, s2d):
    return pl.pallas_call(
        _prep_body,
        grid=(M // _BLK,),
        in_specs=[
            pl.BlockSpec((_BLK, C), lambda i: (i, 0)),
            pl.BlockSpec((M // 128, 128), lambda i: (0, 0)),
            pl.BlockSpec((M // 128, 128), lambda i: (0, 0)),
            pl.BlockSpec((M // 128, 128), lambda i: (0, 0)),
        ],
        out_specs=[
            pl.BlockSpec((C, _BLK), lambda i: (0, i)),
            pl.BlockSpec((M // 128, 128), lambda i: (0, 0)),
            pl.BlockSpec((1, 128), lambda i: (0, 0)),
        ],
        out_shape=[
            jax.ShapeDtypeStruct((C, M), jnp.float32),
            jax.ShapeDtypeStruct((M // 128, 128), jnp.int32),
            jax.ShapeDtypeStruct((1, 128), jnp.int32),
        ],
    )(feats, x2d, y2d, s2d)


@functools.partial(
    pl.kernel,
    mesh=plsc.VectorSubcoreMesh(core_axis_name="c", subcore_axis_name="s"),
    out_type=jax.ShapeDtypeStruct((B, C, NX, NY), jnp.float32),
    compiler_params=pltpu.CompilerParams(
        needs_layout_passes=False, use_tc_tiling_on_sc=True
    ),
    scratch_types=[
        pltpu.VMEM((WCELLS,), jnp.int32),
        pltpu.VMEM((2, STR_ROWS, NY), jnp.float32),
        pltpu.VMEM((1, 128), jnp.int32),
        pltpu.SemaphoreType.DMA,
        pltpu.SemaphoreType.DMA,
        pltpu.SemaphoreType.DMA,
    ],
)
def _sc_scatter(
    featsT_hbm, dst_hbm, cnt_hbm, out_hbm,
    win_v, out_v, cnt_v, sem_f, sem_o0, sem_o1,
):
    wid = lax.axis_index("s") * 2 + lax.axis_index("c")
    b = wid // 8
    j = wid % 8
    rs = j * ROWS_BIG - 8 * jnp.maximum(0, j - 6)
    nrows = ROWS_BIG - 8 * (j >= 6).astype(jnp.int32)
    base = (b * NX + rs) * NY
    ncell = nrows * NY
    nstr = nrows // STR_ROWS

    fill = jnp.full((16,), M, jnp.int32)

    def init_body(i, _):
        for k in range(8):
            win_v[pl.ds((i * 8 + k) * 16, 16)] = fill
        return 0

    lax.fori_loop(0, WCELLS // 128, init_body, 0)

    pltpu.sync_copy(cnt_hbm, cnt_v)
    cvec = cnt_v[0, pl.ds(0, 16)]
    iota = lax.iota(jnp.int32, 16)
    lo = jnp.sum(jnp.where(iota < b, cvec, 0))
    nb = jnp.sum(jnp.where(iota == b, cvec, 0))
    v8_lo = lo // 128
    v8_hi = (lo + nb + 127) // 128

    def phase1(dst_v):
        pltpu.sync_copy(dst_hbm, dst_v)

        def body(v8, _):
            for k in range(8):
                v = v8 * 8 + k
                idx = dst_v[pl.ds(v * 16, 16)]
                m = iota + v * 16
                local = idx - base
                mask = (local >= 0) & (local < ncell)
                safe = jnp.where(mask, local, 0)
                plsc.store_scatter(win_v, [safe], m, mask=mask)
            return 0

        lax.fori_loop(v8_lo, v8_hi, body, 0)

    pl.run_scoped(phase1, pltpu.VMEM((M,), jnp.int32))

    def phase2(feat_v):
      pltpu.make_async_copy(
          featsT_hbm.at[pl.ds(0, FB)], feat_v.at[0], sem_f
      ).start()

      def chan(c, _):
          par = lax.rem(c, 2)
          pltpu.make_async_copy(
              featsT_hbm.at[pl.ds(c * FB, FB)], feat_v.at[par], sem_f
          ).wait()

          @pl.when(c < C - 1)
          def _():
              pltpu.make_async_copy(
                  featsT_hbm.at[pl.ds((c + 1) * FB, FB)], feat_v.at[1 - par], sem_f
              ).start()

          prow = jnp.zeros((16,), jnp.int32) + par

          for st in range(NSTR):
              p = st % 2
              sem_o = sem_o0 if p == 0 else sem_o1

              @pl.when(st < nstr)
              def _():
                  src = out_v.at[p]
                  if st >= 2:
                      pltpu.make_async_copy(
                          src,
                          out_hbm.at[b, c, pl.ds(rs + (st - 2) * STR_ROWS, STR_ROWS), :],
                          sem_o,
                      ).wait()
                  else:

                      @pl.when(c > 0)
                      def _():
                          pltpu.make_async_copy(
                              src, out_hbm.at[b, c, pl.ds(rs, STR_ROWS), :], sem_o
                          ).wait()

                  for r in range(STR_ROWS):
                      lrow = st * STR_ROWS + r

                      @plsc.parallel_loop(0, NY, step=16, unroll=8)
                      def _(i):
                          widx = win_v[pl.ds(lrow * NY + i, 16)]
                          vals = plsc.load_gather(
                              feat_v,
                              [prow, lax.shift_right_logical(widx, 7), widx & 127],
                          )
                          out_v[p, r, pl.ds(i, 16)] = vals

                  pltpu.make_async_copy(
                      src,
                      out_hbm.at[b, c, pl.ds(rs + st * STR_ROWS, STR_ROWS), :],
                      sem_o,
                  ).start()

          return 0

      lax.fori_loop(0, C, chan, 0)

      for p, sem_o in ((0, sem_o0), (1, sem_o1)):
          pltpu.make_async_copy(
              out_v.at[p], out_hbm.at[0, 0, pl.ds(p * STR_ROWS, STR_ROWS), :], sem_o
          ).wait()

    pl.run_scoped(phase2, pltpu.VMEM((2, FB, 128), jnp.float32))


def kernel(batch_pillar_features, batch_indices, sample_indices, batch_size):
    del batch_size
    featsT, dst2d, cnt = _prep(
        batch_pillar_features,
        batch_indices[:, 2].reshape(M // 128, 128),
        batch_indices[:, 1].reshape(M // 128, 128),
        sample_indices.reshape(M // 128, 128),
    )
    featsT_p = jnp.pad(
        featsT.reshape(C, M // 128, 128), ((0, 0), (0, 1), (0, 0))
    ).reshape(C * FB, 128)
    return _sc_scatter(featsT_p, dst2d.reshape(M), cnt)
